# baseline (speedup 1.0000x reference)
import dataclasses

import jax
import jax.numpy as jnp
from jax import lax
from jax.experimental import pallas as pl
from jax.experimental.pallas import tpu as pltpu
from jax.experimental.pallas import tpu_sc as plsc

N = 4194304
NC, NS, L = 2, 16, 16
NW = NC * NS
P = N // NW
B = 1024
CH = P // B
STEPS = B // L
HALF = 8388608

_mesh = plsc.VectorSubcoreMesh(core_axis_name="c", subcore_axis_name="s")

_cp = pltpu.CompilerParams()
if "needs_layout_passes" in pltpu.CompilerParams.__dataclass_fields__:
    _cp = dataclasses.replace(_cp, needs_layout_passes=False)


class _Buf:
    def __init__(self, xv, yv, zv, wx, wy, wz, idx4, gat, outv,
                 semx, semg, semo):
        self.xv, self.yv, self.zv = xv, yv, zv
        self.wx, self.wy, self.wz = wx, wy, wz
        self.idx4, self.gat, self.outv = idx4, gat, outv
        self.semx, self.semg, self.semo = semx, semg, semo


def _body(xs_hbm, tab_hbm, out_hbm, *scratch):
    a = _Buf(*scratch[0:9], scratch[18], scratch[20], scratch[22])
    b = _Buf(*scratch[9:18], scratch[19], scratch[21], scratch[23])
    wid = lax.axis_index("s") * NC + lax.axis_index("c")

    def fire_xs(cg, s):
        base = wid * P + cg * B
        for d, dst in ((0, s.xv), (1, s.yv), (2, s.zv)):
            pltpu.async_copy(xs_hbm.at[pl.ds(d * N + base, B)], dst, s.semx)

    def wait_xs(cg, s):
        base = wid * P + cg * B
        for d, dst in ((0, s.xv), (1, s.yv), (2, s.zv)):
            pltpu.make_async_copy(
                xs_hbm.at[pl.ds(d * N + base, B)], dst, s.semx).wait()

    def idx_pass(s):
        for st in range(STEPS):
            o = st * L
            xf = s.xv[pl.ds(o, L)] * 255.0
            yf = s.yv[pl.ds(o, L)] * 255.0
            zf = s.zv[pl.ds(o, L)] * 255.0
            ix = xf.astype(jnp.int32)
            iy = yf.astype(jnp.int32)
            iz = zf.astype(jnp.int32)
            s.wx[pl.ds(o, L)] = xf - ix.astype(jnp.float32)
            s.wy[pl.ds(o, L)] = yf - iy.astype(jnp.float32)
            s.wz[pl.ds(o, L)] = zf - iz.astype(jnp.float32)
            f000 = (ix << 16) + (iy << 8) + iz
            r00 = (f000 >> 1) + ((f000 & 1) << 23)
            s.idx4[pl.ds(0 * B + o, L)] = r00
            s.idx4[pl.ds(1 * B + o, L)] = r00 + 128
            s.idx4[pl.ds(2 * B + o, L)] = r00 + 32768
            s.idx4[pl.ds(3 * B + o, L)] = r00 + 32896

    def fire_g(s):
        pltpu.async_copy(tab_hbm.at[s.idx4], s.gat, s.semg)

    def wait_g(s):
        pltpu.make_async_copy(tab_hbm.at[s.idx4], s.gat, s.semg).wait()

    def _unpack(w):
        z0 = plsc.bitcast(w << 16, jnp.float32)
        z1 = plsc.bitcast(w & -65536, jnp.float32)
        return z0, z1

    def combine(s):
        for st in range(STEPS):
            o = st * L
            wx = s.wx[pl.ds(o, L)]
            wy = s.wy[pl.ds(o, L)]
            wz = s.wz[pl.ds(o, L)]
            c000, c001 = _unpack(s.gat[pl.ds(0 * B + o, L)])
            c010, c011 = _unpack(s.gat[pl.ds(1 * B + o, L)])
            c100, c101 = _unpack(s.gat[pl.ds(2 * B + o, L)])
            c110, c111 = _unpack(s.gat[pl.ds(3 * B + o, L)])
            c00 = c000 + wz * (c001 - c000)
            c01 = c010 + wz * (c011 - c010)
            c10 = c100 + wz * (c101 - c100)
            c11 = c110 + wz * (c111 - c110)
            c0 = c00 + wy * (c01 - c00)
            c1 = c10 + wy * (c11 - c10)
            s.outv[pl.ds(o, L)] = c0 + wx * (c1 - c0)

    def fire_out(cg, s):
        pltpu.async_copy(s.outv, out_hbm.at[pl.ds(wid * P + cg * B, B)],
                         s.semo)

    def wait_out(s):
        pltpu.make_async_copy(s.outv, out_hbm.at[pl.ds(wid * P, B)],
                              s.semo).wait()

    def half(cg, cur, oth):
        @pl.when(cg + 1 < CH)
        def _():
            fire_xs(cg + 1, oth)

        @pl.when(cg < CH)
        def _():
            wait_xs(cg, cur)
            idx_pass(cur)
            fire_g(cur)

        @pl.when((cg >= 1) & (cg <= CH))
        def _():
            wait_g(oth)

            @pl.when(cg >= 3)
            def _():
                wait_out(oth)

            combine(oth)
            fire_out(cg - 1, oth)

    fire_xs(0, a)

    @pl.loop(0, CH // 2 + 1)
    def _main(m):
        half(2 * m, a, b)
        half(2 * m + 1, b, a)

    wait_out(a)
    wait_out(b)


def kernel(xs, data):
    xs_t = xs.T.reshape(-1)
    f16 = data.reshape(-1).astype(jnp.bfloat16)
    sh16 = jnp.concatenate([f16[1:], jnp.zeros((1,), jnp.bfloat16)])
    tab = jnp.concatenate([
        lax.bitcast_convert_type(f16.reshape(HALF, 2), jnp.int32),
        lax.bitcast_convert_type(sh16.reshape(HALF, 2), jnp.int32),
    ])

    def bufset():
        return [
            pltpu.VMEM((B,), jnp.float32),
            pltpu.VMEM((B,), jnp.float32),
            pltpu.VMEM((B,), jnp.float32),
            pltpu.VMEM((B,), jnp.float32),
            pltpu.VMEM((B,), jnp.float32),
            pltpu.VMEM((B,), jnp.float32),
            pltpu.VMEM((4 * B,), jnp.int32),
            pltpu.VMEM((4 * B,), jnp.int32),
            pltpu.VMEM((B,), jnp.float32),
        ]

    run = pl.kernel(
        _body,
        out_type=jax.ShapeDtypeStruct((N,), jnp.float32),
        mesh=_mesh,
        scratch_types=bufset() + bufset() + [
            pltpu.SemaphoreType.DMA,
            pltpu.SemaphoreType.DMA,
            pltpu.SemaphoreType.DMA,
            pltpu.SemaphoreType.DMA,
            pltpu.SemaphoreType.DMA,
            pltpu.SemaphoreType.DMA,
        ],
        compiler_params=_cp,
    )
    return run(xs_t, tab)


# speedup vs baseline: 8.6170x; 8.6170x over previous
import dataclasses

import jax
import jax.numpy as jnp
from jax import lax
from jax.experimental import pallas as pl
from jax.experimental.pallas import tpu as pltpu
from jax.experimental.pallas import tpu_sc as plsc

N = 4194304
NC, NS, L = 2, 16, 16
NW = NC * NS
P = N // NW
B = 1024
CH = P // B
STEPS = B // L
HALF = 8388608

_mesh = plsc.VectorSubcoreMesh(core_axis_name="c", subcore_axis_name="s")

_cp = pltpu.CompilerParams()
if "needs_layout_passes" in pltpu.CompilerParams.__dataclass_fields__:
    _cp = dataclasses.replace(_cp, needs_layout_passes=False)


class _Buf:
    def __init__(self, xv, yv, zv, wx, wy, wz, idx4, gat, outv,
                 semx, semg, semo):
        self.xv, self.yv, self.zv = xv, yv, zv
        self.wx, self.wy, self.wz = wx, wy, wz
        self.idx4, self.gat, self.outv = idx4, gat, outv
        self.semx, self.semg, self.semo = semx, semg, semo


def _body(xs_hbm, tab_hbm, out_hbm, *scratch):
    a = _Buf(*scratch[0:9], scratch[18], scratch[20], scratch[22])
    b = _Buf(*scratch[9:18], scratch[19], scratch[21], scratch[23])
    wid = lax.axis_index("s") * NC + lax.axis_index("c")

    def fire_xs(cg, s):
        base = wid * P + cg * B
        for d, dst in ((0, s.xv), (1, s.yv), (2, s.zv)):
            pltpu.async_copy(xs_hbm.at[pl.ds(d * N + base, B)], dst, s.semx)

    def wait_xs(cg, s):
        base = wid * P + cg * B
        for d, dst in ((0, s.xv), (1, s.yv), (2, s.zv)):
            pltpu.make_async_copy(
                xs_hbm.at[pl.ds(d * N + base, B)], dst, s.semx).wait()

    def idx_pass(s):
        for st in range(STEPS):
            o = st * L
            xf = s.xv[pl.ds(o, L)] * 255.0
            yf = s.yv[pl.ds(o, L)] * 255.0
            zf = s.zv[pl.ds(o, L)] * 255.0
            ix = xf.astype(jnp.int32)
            iy = yf.astype(jnp.int32)
            iz = zf.astype(jnp.int32)
            s.wx[pl.ds(o, L)] = xf - ix.astype(jnp.float32)
            s.wy[pl.ds(o, L)] = yf - iy.astype(jnp.float32)
            s.wz[pl.ds(o, L)] = zf - iz.astype(jnp.float32)
            f000 = (ix << 16) + (iy << 8) + iz
            s.idx4[pl.ds(0 * B + o, L)] = f000
            s.idx4[pl.ds(1 * B + o, L)] = f000 + 256
            s.idx4[pl.ds(2 * B + o, L)] = f000 + 65536
            s.idx4[pl.ds(3 * B + o, L)] = f000 + 65792

    def fire_g(s):
        pltpu.async_copy(tab_hbm.at[s.idx4], s.gat, s.semg)

    def wait_g(s):
        pltpu.make_async_copy(tab_hbm.at[s.idx4], s.gat, s.semg).wait()

    def _unpack(w):
        z0 = plsc.bitcast(w << 16, jnp.float32)
        z1 = plsc.bitcast(w & -65536, jnp.float32)
        return z0, z1

    def combine(s):
        for st in range(STEPS):
            o = st * L
            wx = s.wx[pl.ds(o, L)]
            wy = s.wy[pl.ds(o, L)]
            wz = s.wz[pl.ds(o, L)]
            c000, c001 = _unpack(s.gat[pl.ds(0 * B + o, L)])
            c010, c011 = _unpack(s.gat[pl.ds(1 * B + o, L)])
            c100, c101 = _unpack(s.gat[pl.ds(2 * B + o, L)])
            c110, c111 = _unpack(s.gat[pl.ds(3 * B + o, L)])
            c00 = c000 + wz * (c001 - c000)
            c01 = c010 + wz * (c011 - c010)
            c10 = c100 + wz * (c101 - c100)
            c11 = c110 + wz * (c111 - c110)
            c0 = c00 + wy * (c01 - c00)
            c1 = c10 + wy * (c11 - c10)
            s.outv[pl.ds(o, L)] = c0 + wx * (c1 - c0)

    def fire_out(cg, s):
        pltpu.async_copy(s.outv, out_hbm.at[pl.ds(wid * P + cg * B, B)],
                         s.semo)

    def wait_out(s):
        pltpu.make_async_copy(s.outv, out_hbm.at[pl.ds(wid * P, B)],
                              s.semo).wait()

    def half(cg, cur, oth):
        @pl.when(cg + 1 < CH)
        def _():
            fire_xs(cg + 1, oth)

        @pl.when(cg < CH)
        def _():
            wait_xs(cg, cur)
            idx_pass(cur)
            fire_g(cur)

        @pl.when((cg >= 1) & (cg <= CH))
        def _():
            wait_g(oth)

            @pl.when(cg >= 3)
            def _():
                wait_out(oth)

            combine(oth)
            fire_out(cg - 1, oth)

    fire_xs(0, a)

    @pl.loop(0, CH // 2 + 1)
    def _main(m):
        half(2 * m, a, b)
        half(2 * m + 1, b, a)

    wait_out(a)
    wait_out(b)


def kernel(xs, data):
    xs_t = xs.T.reshape(-1)
    wi = lax.bitcast_convert_type(data.reshape(-1), jnp.int32)
    rnd = (wi + 32767 + ((wi >> 16) & 1)) >> 16
    nxt = jnp.concatenate([rnd[1:], jnp.zeros((1,), jnp.int32)])
    tab = rnd | (nxt << 16)

    def bufset():
        return [
            pltpu.VMEM((B,), jnp.float32),
            pltpu.VMEM((B,), jnp.float32),
            pltpu.VMEM((B,), jnp.float32),
            pltpu.VMEM((B,), jnp.float32),
            pltpu.VMEM((B,), jnp.float32),
            pltpu.VMEM((B,), jnp.float32),
            pltpu.VMEM((4 * B,), jnp.int32),
            pltpu.VMEM((4 * B,), jnp.int32),
            pltpu.VMEM((B,), jnp.float32),
        ]

    run = pl.kernel(
        _body,
        out_type=jax.ShapeDtypeStruct((N,), jnp.float32),
        mesh=_mesh,
        scratch_types=bufset() + bufset() + [
            pltpu.SemaphoreType.DMA,
            pltpu.SemaphoreType.DMA,
            pltpu.SemaphoreType.DMA,
            pltpu.SemaphoreType.DMA,
            pltpu.SemaphoreType.DMA,
            pltpu.SemaphoreType.DMA,
        ],
        compiler_params=_cp,
    )
    return run(xs_t, tab)
